# baseline (speedup 1.0000x reference)
import functools

import jax
import jax.numpy as jnp
from jax import lax
from jax.experimental import pallas as pl
from jax.experimental.pallas import tpu as pltpu
from jax.experimental.pallas import tpu_sc as plsc

D_MODEL = 64
GATHER_W = 128
CHUNK = 512


def _embed_lookup(xf, table, *, n, num_cores, num_subcores):
    nw = num_cores * num_subcores
    per_w = n // nw
    steps = per_w // CHUNK
    g_per_chunk = CHUNK // GATHER_W

    mesh = plsc.VectorSubcoreMesh(core_axis_name="c", subcore_axis_name="s")

    @functools.partial(
        pl.kernel,
        mesh=mesh,
        compiler_params=pltpu.CompilerParams(use_tc_tiling_on_sc=False),
        out_type=jax.ShapeDtypeStruct((n, D_MODEL), jnp.float32),
        scratch_types=[
            pltpu.VMEM((CHUNK,), jnp.int32),
            pltpu.VMEM((CHUNK, D_MODEL), jnp.float32),
            pltpu.SemaphoreType.DMA,
        ],
    )
    def k(xf_hbm, table_hbm, out_hbm, idx_v, rows_v, sem):
        wid = lax.axis_index("s") * num_cores + lax.axis_index("c")
        base = wid * per_w

        def body(g, carry):
            off = base + g * CHUNK
            pltpu.sync_copy(xf_hbm.at[pl.ds(off, CHUNK)], idx_v)
            handles = []
            for j in range(g_per_chunk):
                h = pltpu.async_copy(
                    table_hbm.at[idx_v.at[pl.ds(j * GATHER_W, GATHER_W)]],
                    rows_v.at[pl.ds(j * GATHER_W, GATHER_W)],
                    sem,
                )
                handles.append(h)
            for h in handles:
                h.wait()
            pltpu.sync_copy(rows_v, out_hbm.at[pl.ds(off, CHUNK)])
            return carry

        lax.fori_loop(0, steps, body, 0)

    return k(xf, table)


def kernel(x, table):
    b, l = x.shape
    n = b * l
    info = plsc.get_sparse_core_info()
    xf = x.reshape(n)
    out = _embed_lookup(
        xf, table, n=n,
        num_cores=info.num_cores, num_subcores=info.num_subcores,
    )
    return out.reshape(b, l, D_MODEL)


# speedup vs baseline: 1.0461x; 1.0461x over previous
import functools

import jax
import jax.numpy as jnp
from jax import lax
from jax.experimental import pallas as pl
from jax.experimental.pallas import tpu as pltpu
from jax.experimental.pallas import tpu_sc as plsc

D_MODEL = 64
GATHER_W = 128
CHUNK = 512
NGATH = CHUNK // GATHER_W


def _embed_lookup(xf, table, *, n, num_cores, num_subcores):
    nw = num_cores * num_subcores
    per_w = n // nw
    steps = per_w // CHUNK
    assert steps % 2 == 0 and steps >= 4

    mesh = plsc.VectorSubcoreMesh(core_axis_name="c", subcore_axis_name="s")

    @functools.partial(
        pl.kernel,
        mesh=mesh,
        compiler_params=pltpu.CompilerParams(use_tc_tiling_on_sc=False),
        out_type=jax.ShapeDtypeStruct((n, D_MODEL), jnp.float32),
        scratch_types=[
            pltpu.VMEM((CHUNK,), jnp.int32),
            pltpu.VMEM((CHUNK,), jnp.int32),
            pltpu.VMEM((CHUNK, D_MODEL), jnp.float32),
            pltpu.VMEM((CHUNK, D_MODEL), jnp.float32),
            pltpu.SemaphoreType.DMA,
            pltpu.SemaphoreType.DMA,
            pltpu.SemaphoreType.DMA,
            pltpu.SemaphoreType.DMA,
            pltpu.SemaphoreType.DMA,
            pltpu.SemaphoreType.DMA,
        ],
    )
    def k(xf_hbm, table_hbm, out_hbm, idx0, idx1, rows0, rows1,
          sem_i0, sem_i1, sem_g0, sem_g1, sem_w0, sem_w1):
        idx = (idx0, idx1)
        rows = (rows0, rows1)
        sem_i = (sem_i0, sem_i1)
        sem_g = (sem_g0, sem_g1)
        sem_w = (sem_w0, sem_w1)

        wid = lax.axis_index("s") * num_cores + lax.axis_index("c")
        base = wid * per_w

        pltpu.async_copy(xf_hbm.at[pl.ds(base, CHUNK)], idx[0], sem_i[0])

        def substep(g, p):
            off = base + g * CHUNK
            @pl.when(g >= 2)
            def _():
                pltpu.make_async_copy(
                    rows[p], out_hbm.at[pl.ds(off, CHUNK)], sem_w[p]).wait()
            pltpu.make_async_copy(
                xf_hbm.at[pl.ds(off, CHUNK)], idx[p], sem_i[p]).wait()
            handles = []
            for j in range(NGATH):
                handles.append(pltpu.async_copy(
                    table_hbm.at[idx[p].at[pl.ds(j * GATHER_W, GATHER_W)]],
                    rows[p].at[pl.ds(j * GATHER_W, GATHER_W)],
                    sem_g[p]))
            @pl.when(g + 1 < steps)
            def _():
                pltpu.async_copy(
                    xf_hbm.at[pl.ds(off + CHUNK, CHUNK)], idx[1 - p],
                    sem_i[1 - p])
            for h in handles:
                h.wait()
            pltpu.async_copy(rows[p], out_hbm.at[pl.ds(off, CHUNK)], sem_w[p])

        def body(i, carry):
            substep(2 * i, 0)
            substep(2 * i + 1, 1)
            return carry

        lax.fori_loop(0, steps // 2, body, 0)

        for p in range(2):
            g = steps - 2 + p
            off = base + g * CHUNK
            pltpu.make_async_copy(
                rows[p], out_hbm.at[pl.ds(off, CHUNK)], sem_w[p]).wait()

    return k(xf, table)


def kernel(x, table):
    b, l = x.shape
    n = b * l
    info = plsc.get_sparse_core_info()
    xf = x.reshape(n)
    out = _embed_lookup(
        xf, table, n=n,
        num_cores=info.num_cores, num_subcores=info.num_subcores,
    )
    return out.reshape(b, l, D_MODEL)
